# baseline (speedup 1.0000x reference)
import jax
import jax.numpy as jnp
import numpy as np
from jax import lax
from jax.experimental import pallas as pl
from jax.experimental.pallas import tpu as pltpu
from jax.experimental.pallas import tpu_sc as plsc

N = 10000
E = 320000
D = 128
DW = D // 2

NC = 2
NS = 16
K = 112
NPAD = 10240
NCH = 96
EPAD = NC * NS * NCH * K
NGRP = NCH // 8
NPAIR = NGRP // 2
ROWS_PER_TILE = NPAD // NS
ZROWS = 64

_PERM = np.zeros((D, D), np.float32)
for _t in range(D // 32):
    for _p in range(32):
        _c = 32 * _t + (2 * _p if _p < 16 else 2 * (_p - 16) + 1)
        _PERM[32 * _t + _p, _c] = 1.0


def _sc_scatter_kernel(x_hbm, src_hbm, dst_hbm, out0_hbm, out1_hbm,
                       pk0, pk1, rowsf, srcg0, srcg1, dstg0, dstg1,
                       acc_shared, gsem, ssem, isrc, idst):
    pks = (pk0, pk1)
    srcg = (srcg0, srcg1)
    dstg = (dstg0, dstg1)
    c = lax.axis_index("c")
    s = lax.axis_index("s")
    chunk0 = (c * NS + s) * NCH

    def issue_idx(goff, b):
        pltpu.async_copy(src_hbm.at[pl.ds(chunk0 + goff, 8)], srcg[b],
                         isrc.at[b])
        pltpu.async_copy(dst_hbm.at[pl.ds(chunk0 + goff, 8)], dstg[b],
                         idst.at[b])

    def wait_idx(b):
        pltpu.make_async_copy(src_hbm.at[pl.ds(chunk0, 8)], srcg[b],
                              isrc.at[b]).wait()
        pltpu.make_async_copy(dst_hbm.at[pl.ds(chunk0, 8)], dstg[b],
                              idst.at[b]).wait()

    def issue_gather(gb, grow, b):
        pltpu.async_copy(x_hbm.at[srcg[gb].at[grow]], pks[b], gsem.at[b])

    def wait_gather(b):
        pltpu.make_async_copy(x_hbm.at[srcg0.at[0]], pks[b],
                              gsem.at[b]).wait()

    def issue_scatter(gb, grow):
        pltpu.async_copy(rowsf, acc_shared.at[dstg[gb].at[grow]], ssem,
                         add=True)

    def wait_scatter():
        pltpu.make_async_copy(rowsf, acc_shared.at[dstg0.at[0]],
                              ssem).wait()

    def convert(b):
        pref = pks[b]

        hi_mask = jnp.int32(-65536)

        def _rows(r, carry):
            for q in range(DW // 16):
                w = pref[r, pl.ds(q * 16, 16)]
                lo = lax.bitcast_convert_type(
                    lax.shift_left(w, 16), jnp.float32)
                hi = lax.bitcast_convert_type(
                    lax.bitwise_and(w, hi_mask), jnp.float32)
                rowsf[r, pl.ds(q * 32, 16)] = lo
                rowsf[r, pl.ds(q * 32 + 16, 16)] = hi
            return carry
        lax.fori_loop(0, K, _rows, 0)

    def _zrow(i, carry):
        for j in range(D // 16):
            rowsf[i, pl.ds(j * 16, 16)] = jnp.zeros((16,), jnp.float32)
        return carry
    lax.fori_loop(0, ZROWS, _zrow, 0)
    zsrc = rowsf.at[pl.ds(0, ZROWS)]
    for r in range(ROWS_PER_TILE // ZROWS):
        pltpu.sync_copy(
            zsrc, acc_shared.at[pl.ds(s * ROWS_PER_TILE + r * ZROWS, ZROWS)])
    plsc.subcore_barrier()

    def emit_pair(p, first_pair, last_pair):
        jbase = 16 * p
        for l in range(16):
            b = l % 2
            if l == 7:
                wait_idx(1)
            if l == 15 and not last_pair:
                wait_idx(0)
            if not (last_pair and l == 15):
                gsel = (l + 1) // 8
                gb = (0, 1, 0)[gsel]
                issue_gather(gb, (l + 1) % 8, 1 - b)
            if l == 0:
                issue_idx(jbase + 8, 1)
            if l == 8 and not last_pair:
                issue_idx(jbase + 16, 0)
            if not (first_pair and l == 0):
                wait_scatter()
            wait_gather(b)
            convert(b)
            issue_scatter(l // 8, l % 8)

    issue_idx(0, 0)
    wait_idx(0)
    issue_gather(0, 0, 0)
    emit_pair(0, True, False)

    def _ring(p, carry):
        emit_pair(p, False, False)
        return carry
    lax.fori_loop(1, NPAIR - 1, _ring, 0)

    emit_pair(NPAIR - 1, False, True)
    wait_scatter()
    plsc.subcore_barrier()

    row0 = s * ROWS_PER_TILE
    acc_slice = acc_shared.at[pl.ds(row0, ROWS_PER_TILE)]

    @pl.when(c == 0)
    def _():
        pltpu.sync_copy(acc_slice, out0_hbm.at[pl.ds(row0, ROWS_PER_TILE)])

    @pl.when(c == 1)
    def _():
        pltpu.sync_copy(acc_slice, out1_hbm.at[pl.ds(row0, ROWS_PER_TILE)])


@jax.jit
def _sc_scatter(x_pk, src2d, dst2d):
    return pl.kernel(
        _sc_scatter_kernel,
        out_type=(jax.ShapeDtypeStruct((NPAD, D), jnp.float32),
                  jax.ShapeDtypeStruct((NPAD, D), jnp.float32)),
        mesh=plsc.VectorSubcoreMesh(core_axis_name="c", subcore_axis_name="s"),
        compiler_params=pltpu.CompilerParams(use_tc_tiling_on_sc=False),
        scratch_types=[
            pltpu.VMEM((K, DW), jnp.int32),
            pltpu.VMEM((K, DW), jnp.int32),
            pltpu.VMEM((K, D), jnp.float32),
            pltpu.VMEM((8, K), jnp.int32),
            pltpu.VMEM((8, K), jnp.int32),
            pltpu.VMEM((8, K), jnp.int32),
            pltpu.VMEM((8, K), jnp.int32),
            pltpu.VMEM_SHARED((NPAD, D), jnp.float32),
            pltpu.SemaphoreType.DMA((2,)),
            pltpu.SemaphoreType.DMA,
            pltpu.SemaphoreType.DMA((2,)),
            pltpu.SemaphoreType.DMA((2,)),
        ],
    )(x_pk, src2d, dst2d)


def _combine_body(p0_ref, p1_ref, x_ref, w_ref, perm_ref, o_ref):
    o_ref[...] = (jnp.dot(p0_ref[...] + p1_ref[...], perm_ref[...],
                          preferred_element_type=jnp.float32)
                  + jnp.dot(x_ref[...], w_ref[...],
                            preferred_element_type=jnp.float32))


@jax.jit
def _tc_combine(p0, p1, x, w):
    blk = 1000
    return pl.pallas_call(
        _combine_body,
        grid=(N // blk,),
        in_specs=[
            pl.BlockSpec((blk, D), lambda i: (i, 0)),
            pl.BlockSpec((blk, D), lambda i: (i, 0)),
            pl.BlockSpec((blk, D), lambda i: (i, 0)),
            pl.BlockSpec((D, D), lambda i: (0, 0)),
            pl.BlockSpec((D, D), lambda i: (0, 0)),
        ],
        out_specs=pl.BlockSpec((blk, D), lambda i: (i, 0)),
        out_shape=jax.ShapeDtypeStruct((N, D), jnp.float32),
    )(p0, p1, x, w, jnp.asarray(_PERM))


def kernel(x, edge_index, loop_weight):
    x_bf = x.astype(jnp.bfloat16)
    x_pk = jax.lax.bitcast_convert_type(
        x_bf.reshape(N, DW, 2), jnp.int32)
    pad = EPAD - E
    src = jnp.concatenate([edge_index[0], jnp.zeros((pad,), jnp.int32)])
    dst = jnp.concatenate([edge_index[1], jnp.full((pad,), N, jnp.int32)])
    src2d = src.reshape(EPAD // K, K)
    dst2d = dst.reshape(EPAD // K, K)
    p0, p1 = _sc_scatter(x_pk, src2d, dst2d)
    return _tc_combine(p0, p1, x, loop_weight)
